# Initial kernel scaffold; baseline (speedup 1.0000x reference)
import jax
import jax.numpy as jnp
from jax.experimental import pallas as pl


def kernel(x, edge_index, W, b, gamma, beta):
    raise NotImplementedError("write your pallas kernel here")


# speedup vs baseline: 15.7742x; 15.7742x over previous
import functools

import jax
import jax.numpy as jnp
from jax import lax
from jax.experimental import pallas as pl
from jax.experimental.pallas import tpu as pltpu
from jax.experimental.pallas import tpu_sc as plsc

N = 10000
D = 256
DH = D // 2
E = 160000
NC, NS, L = 2, 16, 16
N_PAD = 10240
E_PAD = 163840
EPW = E_PAD // NS
CH = 128
NCHUNK = EPW // CH

_mesh = plsc.VectorSubcoreMesh(
    core_axis_name="c", subcore_axis_name="s", num_cores=NC, num_subcores=NS)


@functools.partial(
    pl.kernel,
    out_type=jax.ShapeDtypeStruct((N_PAD,), jnp.float32),
    mesh=_mesh,
    scratch_types=[
        pltpu.VMEM((CH,), jnp.int32),
        pltpu.VMEM((CH,), jnp.float32),
        pltpu.VMEM((640,), jnp.float32),
        pltpu.VMEM_SHARED((N_PAD,), jnp.float32),
    ],
)
def _sc_degree(dst_hbm, deg_out, dstb, onesb, zb, shared):
    c = lax.axis_index("c")
    s = lax.axis_index("s")
    for i in range(640 // L):
        zb[pl.ds(i * L, L)] = jnp.zeros((L,), jnp.float32)
    for i in range(CH // L):
        onesb[pl.ds(i * L, L)] = jnp.ones((L,), jnp.float32)
    pltpu.sync_copy(zb, shared.at[pl.ds(s * 640, 640)])
    plsc.subcore_barrier()

    def body(k, carry):
        eb = s * EPW + k * CH
        pltpu.sync_copy(dst_hbm.at[pl.ds(eb, CH)], dstb)
        pltpu.sync_copy(onesb, shared.at[dstb], add=True)
        return carry

    lax.fori_loop(0, NCHUNK, body, 0)
    plsc.subcore_barrier()
    off = c * (N_PAD // NC) + s * 320
    pltpu.sync_copy(shared.at[pl.ds(off, 320)], zb.at[pl.ds(0, 320)])
    pltpu.sync_copy(zb.at[pl.ds(0, 320)], deg_out.at[pl.ds(off, 320)])


@functools.partial(
    pl.kernel,
    out_type=[
        jax.ShapeDtypeStruct((N_PAD, DH), jnp.float32),
        jax.ShapeDtypeStruct((N_PAD, DH), jnp.float32),
    ],
    mesh=_mesh,
    scratch_types=[
        pltpu.VMEM((CH,), jnp.int32),
        pltpu.VMEM((CH,), jnp.int32),
        pltpu.VMEM((CH,), jnp.int32),
        pltpu.VMEM((CH,), jnp.int32),
        pltpu.VMEM((CH, DH), jnp.float32),
        pltpu.VMEM((CH, DH), jnp.float32),
        pltpu.SemaphoreType.DMA,
        pltpu.SemaphoreType.DMA,
        pltpu.SemaphoreType.DMA,
        pltpu.SemaphoreType.DMA,
        pltpu.VMEM_SHARED((N_PAD, DH), jnp.float32),
    ],
)
def _sc_segsum(hp0_hbm, hp1_hbm, src_hbm, dst_hbm, out0_hbm, out1_hbm,
               srcb0, srcb1, dstb0, dstb1, rows0, rows1,
               gsem0, gsem1, ssem0, ssem1, shared):
    c = lax.axis_index("c")
    s = lax.axis_index("s")
    srcb = (srcb0, srcb1)
    dstb = (dstb0, dstb1)
    rows = (rows0, rows1)
    gsem = (gsem0, gsem1)
    ssem = (ssem0, ssem1)

    def start_gather(b):
        @pl.when(c == 0)
        def _():
            pltpu.async_copy(hp0_hbm.at[srcb[b]], rows[b], gsem[b])

        @pl.when(c == 1)
        def _():
            pltpu.async_copy(hp1_hbm.at[srcb[b]], rows[b], gsem[b])

    def wait_gather(b):
        pltpu.make_async_copy(hp0_hbm.at[srcb[b]], rows[b], gsem[b]).wait()

    def load_idx(k, b):
        eb = s * EPW + k * CH
        pltpu.sync_copy(src_hbm.at[pl.ds(eb, CH)], srcb[b])
        pltpu.sync_copy(dst_hbm.at[pl.ds(eb, CH)], dstb[b])

    def start_scatter(b):
        pltpu.async_copy(rows[b], shared.at[dstb[b]], ssem[b], add=True)

    def wait_scatter(b):
        pltpu.make_async_copy(rows[b], shared.at[dstb[b]], ssem[b]).wait()

    def zbody(i, carry):
        rows0[i // (DH // L), pl.ds((i % (DH // L)) * L, L)] = (
            jnp.zeros((L,), jnp.float32))
        return carry

    lax.fori_loop(0, CH * (DH // L), zbody, 0)
    r0 = s * (N_PAD // NS)
    for j in range(N_PAD // NS // CH):
        pltpu.sync_copy(rows0, shared.at[pl.ds(r0 + j * CH, CH)])
    plsc.subcore_barrier()

    load_idx(0, 0)
    start_gather(0)

    def body(k2, carry):
        @pl.when(k2 > 0)
        def _():
            wait_scatter(1)

        load_idx(2 * k2 + 1, 1)
        start_gather(1)
        wait_gather(0)
        start_scatter(0)

        @pl.when(k2 < NCHUNK // 2 - 1)
        def _():
            wait_scatter(0)
            load_idx(2 * k2 + 2, 0)
            start_gather(0)

        wait_gather(1)
        start_scatter(1)
        return carry

    lax.fori_loop(0, NCHUNK // 2, body, 0)
    wait_scatter(0)
    wait_scatter(1)
    plsc.subcore_barrier()

    for j in range(N_PAD // NS // CH):
        pltpu.sync_copy(shared.at[pl.ds(r0 + j * CH, CH)], rows0)

        @pl.when(c == 0)
        def _():
            pltpu.sync_copy(rows0, out0_hbm.at[pl.ds(r0 + j * CH, CH)])

        @pl.when(c == 1)
        def _():
            pltpu.sync_copy(rows0, out1_hbm.at[pl.ds(r0 + j * CH, CH)])


def _mm_body(x_ref, w_ref, deg_ref, hp0_ref, hp1_ref, dinv_ref):
    di = lax.rsqrt(deg_ref[...] + 1.0)
    h = jnp.dot(x_ref[...], w_ref[...], preferred_element_type=jnp.float32)
    hp = h * di
    hp0_ref[...] = hp[:, :DH]
    hp1_ref[...] = hp[:, DH:]
    dinv_ref[...] = di


BR = 1024
_mm_call = pl.pallas_call(
    _mm_body,
    grid=(N_PAD // BR,),
    in_specs=[
        pl.BlockSpec((BR, D), lambda i: (i, 0)),
        pl.BlockSpec((D, D), lambda i: (0, 0)),
        pl.BlockSpec((BR, 1), lambda i: (i, 0)),
    ],
    out_specs=[
        pl.BlockSpec((BR, DH), lambda i: (i, 0)),
        pl.BlockSpec((BR, DH), lambda i: (i, 0)),
        pl.BlockSpec((BR, 1), lambda i: (i, 0)),
    ],
    out_shape=[
        jax.ShapeDtypeStruct((N_PAD, DH), jnp.float32),
        jax.ShapeDtypeStruct((N_PAD, DH), jnp.float32),
        jax.ShapeDtypeStruct((N_PAD, 1), jnp.float32),
    ],
)


def _final_body(s0_ref, s1_ref, hp0_ref, hp1_ref, dinv_ref, x_ref, b_ref,
                g_ref, be_ref, o_ref):
    sfull = jnp.concatenate([s0_ref[...], s1_ref[...]], axis=1)
    hpfull = jnp.concatenate([hp0_ref[...], hp1_ref[...]], axis=1)
    res = dinv_ref[...] * (sfull + hpfull) + b_ref[...] + x_ref[...]
    mean = jnp.mean(res, axis=-1, keepdims=True)
    cent = res - mean
    var = jnp.mean(cent * cent, axis=-1, keepdims=True)
    o = cent * lax.rsqrt(var + 1e-5) * g_ref[...] + be_ref[...]
    o_ref[...] = jnp.maximum(o, 0.0)


_final_call = pl.pallas_call(
    _final_body,
    grid=(N_PAD // BR,),
    in_specs=[
        pl.BlockSpec((BR, DH), lambda i: (i, 0)),
        pl.BlockSpec((BR, DH), lambda i: (i, 0)),
        pl.BlockSpec((BR, DH), lambda i: (i, 0)),
        pl.BlockSpec((BR, DH), lambda i: (i, 0)),
        pl.BlockSpec((BR, 1), lambda i: (i, 0)),
        pl.BlockSpec((BR, D), lambda i: (i, 0)),
        pl.BlockSpec((1, D), lambda i: (0, 0)),
        pl.BlockSpec((1, D), lambda i: (0, 0)),
        pl.BlockSpec((1, D), lambda i: (0, 0)),
    ],
    out_specs=pl.BlockSpec((BR, D), lambda i: (i, 0)),
    out_shape=jax.ShapeDtypeStruct((N_PAD, D), jnp.float32),
)


def kernel(x, edge_index, W, b, gamma, beta):
    ei = edge_index.astype(jnp.int32)
    pad = E_PAD - E
    pad_rows = N + (jnp.arange(pad, dtype=jnp.int32) % (N_PAD - N))
    src_p = jnp.concatenate([ei[0], pad_rows])
    dst_p = jnp.concatenate([ei[1], pad_rows])

    deg = _sc_degree(dst_p)
    x_p = jnp.pad(x, ((0, N_PAD - N), (0, 0)))
    hp0, hp1, dinv = _mm_call(x_p, W, deg.reshape(N_PAD, 1))
    s0, s1 = _sc_segsum(hp0, hp1, src_p, dst_p)
    out = _final_call(s0, s1, hp0, hp1, dinv, x_p, b.reshape(1, D),
                      gamma.reshape(1, D), beta.reshape(1, D))
    return out[:N]


# speedup vs baseline: 20.3519x; 1.2902x over previous
import functools

import jax
import jax.numpy as jnp
from jax import lax
from jax.experimental import pallas as pl
from jax.experimental.pallas import tpu as pltpu
from jax.experimental.pallas import tpu_sc as plsc

N = 10000
D = 256
DH = D // 2
E = 160000
NC, NS, L = 2, 16, 16
N_PAD = 10240
E_PAD = 163840
EPW = E_PAD // NS
CH = 128
NCHUNK = EPW // CH

_mesh = plsc.VectorSubcoreMesh(
    core_axis_name="c", subcore_axis_name="s", num_cores=NC, num_subcores=NS)


@functools.partial(
    pl.kernel,
    out_type=jax.ShapeDtypeStruct((N_PAD,), jnp.float32),
    mesh=_mesh,
    scratch_types=[
        pltpu.VMEM((NCHUNK, CH), jnp.int32),
        pltpu.VMEM((CH,), jnp.float32),
        pltpu.VMEM((640,), jnp.float32),
        pltpu.SemaphoreType.DMA,
        pltpu.VMEM_SHARED((N_PAD,), jnp.float32),
    ],
)
def _sc_degree(dst_hbm, deg_out, dstb, onesb, zb, sem, shared):
    c = lax.axis_index("c")
    s = lax.axis_index("s")
    for i in range(640 // L):
        zb[pl.ds(i * L, L)] = jnp.zeros((L,), jnp.float32)
    for i in range(CH // L):
        onesb[pl.ds(i * L, L)] = jnp.ones((L,), jnp.float32)
    pltpu.sync_copy(dst_hbm.at[s], dstb)
    pltpu.sync_copy(zb, shared.at[pl.ds(s * 640, 640)])
    plsc.subcore_barrier()

    def body(k, carry):
        pltpu.async_copy(onesb, shared.at[dstb.at[k]], sem, add=True)
        return carry

    lax.fori_loop(0, NCHUNK, body, 0)

    def drain(k, carry):
        pltpu.make_async_copy(onesb, shared.at[dstb.at[k]], sem).wait()
        return carry

    lax.fori_loop(0, NCHUNK, drain, 0)
    plsc.subcore_barrier()
    off = c * (N_PAD // NC) + s * 320
    pltpu.sync_copy(shared.at[pl.ds(off, 320)], zb.at[pl.ds(0, 320)])
    pltpu.sync_copy(zb.at[pl.ds(0, 320)], deg_out.at[pl.ds(off, 320)])


@functools.partial(
    pl.kernel,
    out_type=[
        jax.ShapeDtypeStruct((N_PAD, DH), jnp.float32),
        jax.ShapeDtypeStruct((N_PAD, DH), jnp.float32),
    ],
    mesh=_mesh,
    scratch_types=[
        pltpu.VMEM((CH,), jnp.int32),
        pltpu.VMEM((CH,), jnp.int32),
        pltpu.VMEM((CH,), jnp.int32),
        pltpu.VMEM((CH,), jnp.int32),
        pltpu.VMEM((CH, DH), jnp.float32),
        pltpu.VMEM((CH, DH), jnp.float32),
        pltpu.SemaphoreType.DMA,
        pltpu.SemaphoreType.DMA,
        pltpu.SemaphoreType.DMA,
        pltpu.SemaphoreType.DMA,
        pltpu.SemaphoreType.DMA,
        pltpu.SemaphoreType.DMA,
        pltpu.VMEM_SHARED((N_PAD, DH), jnp.float32),
    ],
)
def _sc_segsum(hp0_hbm, hp1_hbm, src_hbm, dst_hbm, out0_hbm, out1_hbm,
               srcb0, srcb1, dstb0, dstb1, rows0, rows1,
               isem0, isem1, gsem0, gsem1, ssem0, ssem1, shared):
    c = lax.axis_index("c")
    s = lax.axis_index("s")
    srcb = (srcb0, srcb1)
    dstb = (dstb0, dstb1)
    rows = (rows0, rows1)
    isem = (isem0, isem1)
    gsem = (gsem0, gsem1)
    ssem = (ssem0, ssem1)

    def start_load_idx(k, b):
        pltpu.async_copy(src_hbm.at[s, k], srcb[b], isem[b])
        pltpu.async_copy(dst_hbm.at[s, k], dstb[b], isem[b])

    def wait_load_idx(b):
        pltpu.make_async_copy(src_hbm.at[s, 0], srcb[b], isem[b]).wait()
        pltpu.make_async_copy(dst_hbm.at[s, 0], dstb[b], isem[b]).wait()

    def start_gather(b):
        @pl.when(c == 0)
        def _():
            pltpu.async_copy(hp0_hbm.at[srcb[b]], rows[b], gsem[b])

        @pl.when(c == 1)
        def _():
            pltpu.async_copy(hp1_hbm.at[srcb[b]], rows[b], gsem[b])

    def wait_gather(b):
        pltpu.make_async_copy(hp0_hbm.at[srcb[b]], rows[b], gsem[b]).wait()

    def start_scatter(b):
        pltpu.async_copy(rows[b], shared.at[dstb[b]], ssem[b], add=True)

    def wait_scatter(b):
        pltpu.make_async_copy(rows[b], shared.at[dstb[b]], ssem[b]).wait()

    def zbody(i, carry):
        rows0[i // (DH // L), pl.ds((i % (DH // L)) * L, L)] = (
            jnp.zeros((L,), jnp.float32))
        return carry

    lax.fori_loop(0, CH * (DH // L), zbody, 0)
    r0 = s * (N_PAD // NS)
    for j in range(N_PAD // NS // CH):
        pltpu.sync_copy(rows0, shared.at[pl.ds(r0 + j * CH, CH)])
    plsc.subcore_barrier()

    start_load_idx(0, 0)
    wait_load_idx(0)
    start_gather(0)

    def body(k2, carry):
        @pl.when(k2 > 0)
        def _():
            wait_scatter(1)

        start_load_idx(2 * k2 + 1, 1)
        wait_gather(0)
        start_scatter(0)
        wait_load_idx(1)
        start_gather(1)

        @pl.when(k2 < NCHUNK // 2 - 1)
        def _():
            wait_scatter(0)
            start_load_idx(2 * k2 + 2, 0)
            wait_gather(1)
            start_scatter(1)
            wait_load_idx(0)
            start_gather(0)

        @pl.when(k2 == NCHUNK // 2 - 1)
        def _():
            wait_gather(1)
            start_scatter(1)

        return carry

    lax.fori_loop(0, NCHUNK // 2, body, 0)
    wait_scatter(0)
    wait_scatter(1)
    plsc.subcore_barrier()

    for j in range(N_PAD // NS // CH):
        pltpu.sync_copy(shared.at[pl.ds(r0 + j * CH, CH)], rows0)

        @pl.when(c == 0)
        def _():
            pltpu.sync_copy(rows0, out0_hbm.at[pl.ds(r0 + j * CH, CH)])

        @pl.when(c == 1)
        def _():
            pltpu.sync_copy(rows0, out1_hbm.at[pl.ds(r0 + j * CH, CH)])


def _mm_body(x_ref, w_ref, deg_ref, hp0_ref, hp1_ref, dinv_ref):
    di = lax.rsqrt(deg_ref[...] + 1.0)
    h = jnp.dot(x_ref[...], w_ref[...], preferred_element_type=jnp.float32)
    hp = h * di
    hp0_ref[...] = hp[:, :DH]
    hp1_ref[...] = hp[:, DH:]
    dinv_ref[...] = di


BR = 1024
_mm_call = pl.pallas_call(
    _mm_body,
    grid=(N_PAD // BR,),
    in_specs=[
        pl.BlockSpec((BR, D), lambda i: (i, 0)),
        pl.BlockSpec((D, D), lambda i: (0, 0)),
        pl.BlockSpec((BR, 1), lambda i: (i, 0)),
    ],
    out_specs=[
        pl.BlockSpec((BR, DH), lambda i: (i, 0)),
        pl.BlockSpec((BR, DH), lambda i: (i, 0)),
        pl.BlockSpec((BR, 1), lambda i: (i, 0)),
    ],
    out_shape=[
        jax.ShapeDtypeStruct((N_PAD, DH), jnp.float32),
        jax.ShapeDtypeStruct((N_PAD, DH), jnp.float32),
        jax.ShapeDtypeStruct((N_PAD, 1), jnp.float32),
    ],
)


def _final_body(s0_ref, s1_ref, hp0_ref, hp1_ref, dinv_ref, x_ref, b_ref,
                g_ref, be_ref, o_ref):
    sfull = jnp.concatenate([s0_ref[...], s1_ref[...]], axis=1)
    hpfull = jnp.concatenate([hp0_ref[...], hp1_ref[...]], axis=1)
    res = dinv_ref[...] * (sfull + hpfull) + b_ref[...] + x_ref[...]
    mean = jnp.mean(res, axis=-1, keepdims=True)
    cent = res - mean
    var = jnp.mean(cent * cent, axis=-1, keepdims=True)
    o = cent * lax.rsqrt(var + 1e-5) * g_ref[...] + be_ref[...]
    o_ref[...] = jnp.maximum(o, 0.0)


_final_call = pl.pallas_call(
    _final_body,
    grid=(N_PAD // BR,),
    in_specs=[
        pl.BlockSpec((BR, DH), lambda i: (i, 0)),
        pl.BlockSpec((BR, DH), lambda i: (i, 0)),
        pl.BlockSpec((BR, DH), lambda i: (i, 0)),
        pl.BlockSpec((BR, DH), lambda i: (i, 0)),
        pl.BlockSpec((BR, 1), lambda i: (i, 0)),
        pl.BlockSpec((BR, D), lambda i: (i, 0)),
        pl.BlockSpec((1, D), lambda i: (0, 0)),
        pl.BlockSpec((1, D), lambda i: (0, 0)),
        pl.BlockSpec((1, D), lambda i: (0, 0)),
    ],
    out_specs=pl.BlockSpec((BR, D), lambda i: (i, 0)),
    out_shape=jax.ShapeDtypeStruct((N_PAD, D), jnp.float32),
)


def kernel(x, edge_index, W, b, gamma, beta):
    ei = edge_index.astype(jnp.int32)
    pad = E_PAD - E
    pad_rows = N + (jnp.arange(pad, dtype=jnp.int32) % (N_PAD - N))
    src_p = jnp.concatenate([ei[0], pad_rows]).reshape(NS, NCHUNK, CH)
    dst_p = jnp.concatenate([ei[1], pad_rows]).reshape(NS, NCHUNK, CH)

    deg = _sc_degree(dst_p)
    x_p = jnp.pad(x, ((0, N_PAD - N), (0, 0)))
    hp0, hp1, dinv = _mm_call(x_p, W, deg.reshape(N_PAD, 1))
    s0, s1 = _sc_segsum(hp0, hp1, src_p, dst_p)
    out = _final_call(s0, s1, hp0, hp1, dinv, x_p, b.reshape(1, D),
                      gamma.reshape(1, D), beta.reshape(1, D))
    return out[:N]


# speedup vs baseline: 21.1540x; 1.0394x over previous
import functools

import jax
import jax.numpy as jnp
from jax import lax
from jax.experimental import pallas as pl
from jax.experimental.pallas import tpu as pltpu
from jax.experimental.pallas import tpu_sc as plsc

N = 10000
D = 256
DH = D // 2
E = 160000
NC, NS, L = 2, 16, 16
N_PAD = 10240
E_PAD = 163840
EPW = E_PAD // NS
CH = 128
NCHUNK = EPW // CH

_mesh = plsc.VectorSubcoreMesh(
    core_axis_name="c", subcore_axis_name="s", num_cores=NC, num_subcores=NS)


@functools.partial(
    pl.kernel,
    out_type=jax.ShapeDtypeStruct((N_PAD,), jnp.float32),
    mesh=_mesh,
    scratch_types=[
        pltpu.VMEM((NCHUNK, CH), jnp.int32),
        pltpu.VMEM((CH,), jnp.float32),
        pltpu.VMEM((640,), jnp.float32),
        pltpu.SemaphoreType.DMA,
        pltpu.VMEM_SHARED((N_PAD,), jnp.float32),
    ],
)
def _sc_degree(dst_hbm, deg_out, dstb, onesb, zb, sem, shared):
    c = lax.axis_index("c")
    s = lax.axis_index("s")
    for i in range(640 // L):
        zb[pl.ds(i * L, L)] = jnp.zeros((L,), jnp.float32)
    for i in range(CH // L):
        onesb[pl.ds(i * L, L)] = jnp.ones((L,), jnp.float32)
    pltpu.sync_copy(dst_hbm.at[s], dstb)
    pltpu.sync_copy(zb, shared.at[pl.ds(s * 640, 640)])
    plsc.subcore_barrier()

    def body(k, carry):
        pltpu.async_copy(onesb, shared.at[dstb.at[k]], sem, add=True)
        return carry

    lax.fori_loop(0, NCHUNK, body, 0)

    def drain(k, carry):
        pltpu.make_async_copy(onesb, shared.at[dstb.at[k]], sem).wait()
        return carry

    lax.fori_loop(0, NCHUNK, drain, 0)
    plsc.subcore_barrier()
    off = c * (N_PAD // NC) + s * 320
    pltpu.sync_copy(shared.at[pl.ds(off, 320)], zb.at[pl.ds(0, 320)])
    pltpu.sync_copy(zb.at[pl.ds(0, 320)], deg_out.at[pl.ds(off, 320)])


@functools.partial(
    pl.kernel,
    out_type=[
        jax.ShapeDtypeStruct((N_PAD, DH), jnp.float32),
        jax.ShapeDtypeStruct((N_PAD, DH), jnp.float32),
    ],
    mesh=_mesh,
    scratch_types=[
        pltpu.VMEM((CH,), jnp.int32),
        pltpu.VMEM((CH,), jnp.int32),
        pltpu.VMEM((CH,), jnp.int32),
        pltpu.VMEM((CH,), jnp.int32),
        pltpu.VMEM((CH, DH), jnp.float32),
        pltpu.VMEM((CH, DH), jnp.float32),
        pltpu.SemaphoreType.DMA,
        pltpu.SemaphoreType.DMA,
        pltpu.SemaphoreType.DMA,
        pltpu.SemaphoreType.DMA,
        pltpu.SemaphoreType.DMA,
        pltpu.SemaphoreType.DMA,
        pltpu.VMEM_SHARED((N_PAD, DH), jnp.float32),
    ],
)
def _sc_segsum(hp0_hbm, hp1_hbm, src_hbm, dst_hbm, out0_hbm, out1_hbm,
               srcb0, srcb1, dstb0, dstb1, rows0, rows1,
               isem0, isem1, gsem0, gsem1, ssem0, ssem1, shared):
    c = lax.axis_index("c")
    s = lax.axis_index("s")
    srcb = (srcb0, srcb1)
    dstb = (dstb0, dstb1)
    rows = (rows0, rows1)
    isem = (isem0, isem1)
    gsem = (gsem0, gsem1)
    ssem = (ssem0, ssem1)

    def start_load_idx(k, b):
        pltpu.async_copy(src_hbm.at[s, k], srcb[b], isem[b])
        pltpu.async_copy(dst_hbm.at[s, k], dstb[b], isem[b])

    def wait_load_idx(b):
        pltpu.make_async_copy(src_hbm.at[s, 0], srcb[b], isem[b]).wait()
        pltpu.make_async_copy(dst_hbm.at[s, 0], dstb[b], isem[b]).wait()

    def start_gather(b):
        @pl.when(c == 0)
        def _():
            pltpu.async_copy(hp0_hbm.at[srcb[b]], rows[b], gsem[b])

        @pl.when(c == 1)
        def _():
            pltpu.async_copy(hp1_hbm.at[srcb[b]], rows[b], gsem[b])

    def wait_gather(b):
        pltpu.make_async_copy(hp0_hbm.at[srcb[b]], rows[b], gsem[b]).wait()

    def start_scatter(b):
        pltpu.async_copy(rows[b], shared.at[dstb[b]], ssem[b], add=True)

    def wait_scatter(b):
        pltpu.make_async_copy(rows[b], shared.at[dstb[b]], ssem[b]).wait()

    def zbody(i, carry):
        rows0[i // (DH // L), pl.ds((i % (DH // L)) * L, L)] = (
            jnp.zeros((L,), jnp.float32))
        return carry

    lax.fori_loop(0, CH * (DH // L), zbody, 0)
    r0 = s * (N_PAD // NS)
    for j in range(N_PAD // NS // CH):
        pltpu.sync_copy(rows0, shared.at[pl.ds(r0 + j * CH, CH)])
    plsc.subcore_barrier()

    start_load_idx(0, 0)
    wait_load_idx(0)
    start_gather(0)

    def body(k2, carry):
        @pl.when(k2 > 0)
        def _():
            wait_scatter(1)

        start_load_idx(2 * k2 + 1, 1)
        wait_gather(0)
        start_scatter(0)
        wait_load_idx(1)
        start_gather(1)

        @pl.when(k2 < NCHUNK // 2 - 1)
        def _():
            wait_scatter(0)
            start_load_idx(2 * k2 + 2, 0)
            wait_gather(1)
            start_scatter(1)
            wait_load_idx(0)
            start_gather(0)

        @pl.when(k2 == NCHUNK // 2 - 1)
        def _():
            wait_gather(1)
            start_scatter(1)

        return carry

    lax.fori_loop(0, NCHUNK // 2, body, 0)
    wait_scatter(0)
    wait_scatter(1)
    plsc.subcore_barrier()

    for j in range(N_PAD // NS // CH):
        pltpu.sync_copy(shared.at[pl.ds(r0 + j * CH, CH)], rows0)

        @pl.when(c == 0)
        def _():
            pltpu.sync_copy(rows0, out0_hbm.at[pl.ds(r0 + j * CH, CH)])

        @pl.when(c == 1)
        def _():
            pltpu.sync_copy(rows0, out1_hbm.at[pl.ds(r0 + j * CH, CH)])


def _mm_body(x_ref, w_ref, deg_ref, hp0_ref, hp1_ref, dinv_ref):
    di = lax.rsqrt(deg_ref[...] + 1.0)
    h = jnp.dot(x_ref[...], w_ref[...], preferred_element_type=jnp.float32)
    hp = h * di
    hp0_ref[...] = hp[:, :DH]
    hp1_ref[...] = hp[:, DH:]
    dinv_ref[...] = di


BR = 1000
_mm_call = pl.pallas_call(
    _mm_body,
    grid=(N // BR,),
    in_specs=[
        pl.BlockSpec((BR, D), lambda i: (i, 0)),
        pl.BlockSpec((D, D), lambda i: (0, 0)),
        pl.BlockSpec((BR, 1), lambda i: (i, 0)),
    ],
    out_specs=[
        pl.BlockSpec((BR, DH), lambda i: (i, 0)),
        pl.BlockSpec((BR, DH), lambda i: (i, 0)),
        pl.BlockSpec((BR, 1), lambda i: (i, 0)),
    ],
    out_shape=[
        jax.ShapeDtypeStruct((N, DH), jnp.float32),
        jax.ShapeDtypeStruct((N, DH), jnp.float32),
        jax.ShapeDtypeStruct((N, 1), jnp.float32),
    ],
)


def _final_body(s0_ref, s1_ref, hp0_ref, hp1_ref, dinv_ref, x_ref, b_ref,
                g_ref, be_ref, o_ref):
    sfull = jnp.concatenate([s0_ref[...], s1_ref[...]], axis=1)
    hpfull = jnp.concatenate([hp0_ref[...], hp1_ref[...]], axis=1)
    res = dinv_ref[...] * (sfull + hpfull) + b_ref[...] + x_ref[...]
    mean = jnp.mean(res, axis=-1, keepdims=True)
    cent = res - mean
    var = jnp.mean(cent * cent, axis=-1, keepdims=True)
    o = cent * lax.rsqrt(var + 1e-5) * g_ref[...] + be_ref[...]
    o_ref[...] = jnp.maximum(o, 0.0)


_final_call = pl.pallas_call(
    _final_body,
    grid=(N // BR,),
    in_specs=[
        pl.BlockSpec((BR, DH), lambda i: (i, 0)),
        pl.BlockSpec((BR, DH), lambda i: (i, 0)),
        pl.BlockSpec((BR, DH), lambda i: (i, 0)),
        pl.BlockSpec((BR, DH), lambda i: (i, 0)),
        pl.BlockSpec((BR, 1), lambda i: (i, 0)),
        pl.BlockSpec((BR, D), lambda i: (i, 0)),
        pl.BlockSpec((1, D), lambda i: (0, 0)),
        pl.BlockSpec((1, D), lambda i: (0, 0)),
        pl.BlockSpec((1, D), lambda i: (0, 0)),
    ],
    out_specs=pl.BlockSpec((BR, D), lambda i: (i, 0)),
    out_shape=jax.ShapeDtypeStruct((N, D), jnp.float32),
)


def kernel(x, edge_index, W, b, gamma, beta):
    ei = edge_index.astype(jnp.int32)
    pad = E_PAD - E
    arange_pad = jnp.arange(pad, dtype=jnp.int32)
    pad_src = (arange_pad * 613) % N
    pad_dst = N + arange_pad % (N_PAD - N)
    src_p = jnp.concatenate([ei[0], pad_src]).reshape(NS, NCHUNK, CH)
    dst_p = jnp.concatenate([ei[1], pad_dst]).reshape(NS, NCHUNK, CH)

    deg = _sc_degree(dst_p)
    hp0, hp1, dinv = _mm_call(x, W, deg.reshape(N_PAD, 1)[:N])
    s0, s1 = _sc_segsum(hp0, hp1, src_p, dst_p)
    return _final_call(s0, s1, hp0, hp1, dinv, x, b.reshape(1, D),
                       gamma.reshape(1, D), beta.reshape(1, D))


# speedup vs baseline: 23.5590x; 1.1137x over previous
import functools

import jax
import jax.numpy as jnp
from jax import lax
from jax.experimental import pallas as pl
from jax.experimental.pallas import tpu as pltpu
from jax.experimental.pallas import tpu_sc as plsc

N = 10000
D = 256
DH = D // 2
E = 160000
NC, NS, L = 2, 16, 16
N_PAD = 10240
E_PAD = 163840
EPW = E_PAD // NS
CH = 64
NCHUNK = EPW // CH
NB = 4

_mesh = plsc.VectorSubcoreMesh(
    core_axis_name="c", subcore_axis_name="s", num_cores=NC, num_subcores=NS)


@functools.partial(
    pl.kernel,
    out_type=jax.ShapeDtypeStruct((N_PAD,), jnp.float32),
    mesh=_mesh,
    scratch_types=[
        pltpu.VMEM((NCHUNK, CH), jnp.int32),
        pltpu.VMEM((CH,), jnp.float32),
        pltpu.VMEM((640,), jnp.float32),
        pltpu.SemaphoreType.DMA,
        pltpu.VMEM_SHARED((N_PAD,), jnp.float32),
    ],
)
def _sc_degree(dst_hbm, deg_out, dstb, onesb, zb, sem, shared):
    c = lax.axis_index("c")
    s = lax.axis_index("s")
    for i in range(640 // L):
        zb[pl.ds(i * L, L)] = jnp.zeros((L,), jnp.float32)
    for i in range(CH // L):
        onesb[pl.ds(i * L, L)] = jnp.ones((L,), jnp.float32)
    pltpu.sync_copy(dst_hbm.at[s], dstb)
    pltpu.sync_copy(zb, shared.at[pl.ds(s * 640, 640)])
    plsc.subcore_barrier()

    def body(k, carry):
        pltpu.async_copy(onesb, shared.at[dstb.at[k]], sem, add=True)
        return carry

    lax.fori_loop(0, NCHUNK, body, 0)

    def drain(k, carry):
        pltpu.make_async_copy(onesb, shared.at[dstb.at[k]], sem).wait()
        return carry

    lax.fori_loop(0, NCHUNK, drain, 0)
    plsc.subcore_barrier()
    off = c * (N_PAD // NC) + s * 320
    pltpu.sync_copy(shared.at[pl.ds(off, 320)], zb.at[pl.ds(0, 320)])
    pltpu.sync_copy(zb.at[pl.ds(0, 320)], deg_out.at[pl.ds(off, 320)])


@functools.partial(
    pl.kernel,
    out_type=[
        jax.ShapeDtypeStruct((N_PAD, DH), jnp.float32),
        jax.ShapeDtypeStruct((N_PAD, DH), jnp.float32),
    ],
    mesh=_mesh,
    scratch_types=(
        [pltpu.VMEM((CH,), jnp.int32)] * NB
        + [pltpu.VMEM((CH,), jnp.int32)] * NB
        + [pltpu.VMEM((CH, DH), jnp.float32)] * NB
        + [pltpu.SemaphoreType.DMA] * (3 * NB)
        + [pltpu.VMEM_SHARED((N_PAD, DH), jnp.float32)]
    ),
)
def _sc_segsum(hp0_hbm, hp1_hbm, src_hbm, dst_hbm, out0_hbm, out1_hbm,
               sb0, sb1, sb2, sb3, db0, db1, db2, db3, rw0, rw1, rw2, rw3,
               is0, is1, is2, is3, gs0, gs1, gs2, gs3, ss0, ss1, ss2, ss3,
               shared):
    c = lax.axis_index("c")
    s = lax.axis_index("s")
    srcb = (sb0, sb1, sb2, sb3)
    dstb = (db0, db1, db2, db3)
    rows = (rw0, rw1, rw2, rw3)
    isem = (is0, is1, is2, is3)
    gsem = (gs0, gs1, gs2, gs3)
    ssem = (ss0, ss1, ss2, ss3)
    rows0 = rw0

    def start_load_idx(k, b):
        pltpu.async_copy(src_hbm.at[s, k], srcb[b], isem[b])
        pltpu.async_copy(dst_hbm.at[s, k], dstb[b], isem[b])

    def wait_load_idx(b):
        pltpu.make_async_copy(src_hbm.at[s, 0], srcb[b], isem[b]).wait()
        pltpu.make_async_copy(dst_hbm.at[s, 0], dstb[b], isem[b]).wait()

    def start_gather(b):
        @pl.when(c == 0)
        def _():
            pltpu.async_copy(hp0_hbm.at[srcb[b]], rows[b], gsem[b])

        @pl.when(c == 1)
        def _():
            pltpu.async_copy(hp1_hbm.at[srcb[b]], rows[b], gsem[b])

    def wait_gather(b):
        pltpu.make_async_copy(hp0_hbm.at[srcb[b]], rows[b], gsem[b]).wait()

    def start_scatter(b):
        pltpu.async_copy(rows[b], shared.at[dstb[b]], ssem[b], add=True)

    def wait_scatter(b):
        pltpu.make_async_copy(rows[b], shared.at[dstb[b]], ssem[b]).wait()

    def zbody(i, carry):
        rows0[i // (DH // L), pl.ds((i % (DH // L)) * L, L)] = (
            jnp.zeros((L,), jnp.float32))
        return carry

    lax.fori_loop(0, CH * (DH // L), zbody, 0)
    r0 = s * (N_PAD // NS)
    for j in range(N_PAD // NS // CH):
        pltpu.sync_copy(rows0, shared.at[pl.ds(r0 + j * CH, CH)])
    plsc.subcore_barrier()

    LAG = NB - 1

    def body(k0, carry):
        for b in range(NB):
            k = NB * k0 + b

            @pl.when(k0 > 0)
            def _(b=b):
                wait_scatter(b)

            start_load_idx(k, b)

            bc = (b + 1) % NB

            @pl.when(k >= LAG)
            def _(bc=bc):
                wait_gather(bc)
                start_scatter(bc)

            wait_load_idx(b)
            start_gather(b)
        return carry

    lax.fori_loop(0, NCHUNK // NB, body, 0)
    for kc in range(NCHUNK - LAG, NCHUNK):
        wait_gather(kc % NB)
        start_scatter(kc % NB)
    for b in range(NB):
        wait_scatter(b)
    plsc.subcore_barrier()

    for j in range(N_PAD // NS // CH):
        pltpu.sync_copy(shared.at[pl.ds(r0 + j * CH, CH)], rows0)

        @pl.when(c == 0)
        def _():
            pltpu.sync_copy(rows0, out0_hbm.at[pl.ds(r0 + j * CH, CH)])

        @pl.when(c == 1)
        def _():
            pltpu.sync_copy(rows0, out1_hbm.at[pl.ds(r0 + j * CH, CH)])


def _mm_body(x_ref, w_ref, deg_ref, hp0_ref, hp1_ref, dinv_ref):
    di = lax.rsqrt(deg_ref[...] + 1.0)
    h = jnp.dot(x_ref[...], w_ref[...], preferred_element_type=jnp.float32)
    hp = h * di
    hp0_ref[...] = hp[:, :DH]
    hp1_ref[...] = hp[:, DH:]
    dinv_ref[...] = di


BR = 1000
_mm_call = pl.pallas_call(
    _mm_body,
    grid=(N // BR,),
    in_specs=[
        pl.BlockSpec((BR, D), lambda i: (i, 0)),
        pl.BlockSpec((D, D), lambda i: (0, 0)),
        pl.BlockSpec((BR, 1), lambda i: (i, 0)),
    ],
    out_specs=[
        pl.BlockSpec((BR, DH), lambda i: (i, 0)),
        pl.BlockSpec((BR, DH), lambda i: (i, 0)),
        pl.BlockSpec((BR, 1), lambda i: (i, 0)),
    ],
    out_shape=[
        jax.ShapeDtypeStruct((N, DH), jnp.float32),
        jax.ShapeDtypeStruct((N, DH), jnp.float32),
        jax.ShapeDtypeStruct((N, 1), jnp.float32),
    ],
)


def _final_body(s0_ref, s1_ref, hp0_ref, hp1_ref, dinv_ref, x_ref, b_ref,
                g_ref, be_ref, o_ref):
    sfull = jnp.concatenate([s0_ref[...], s1_ref[...]], axis=1)
    hpfull = jnp.concatenate([hp0_ref[...], hp1_ref[...]], axis=1)
    res = dinv_ref[...] * (sfull + hpfull) + b_ref[...] + x_ref[...]
    mean = jnp.mean(res, axis=-1, keepdims=True)
    cent = res - mean
    var = jnp.mean(cent * cent, axis=-1, keepdims=True)
    o = cent * lax.rsqrt(var + 1e-5) * g_ref[...] + be_ref[...]
    o_ref[...] = jnp.maximum(o, 0.0)


_final_call = pl.pallas_call(
    _final_body,
    grid=(N // BR,),
    in_specs=[
        pl.BlockSpec((BR, DH), lambda i: (i, 0)),
        pl.BlockSpec((BR, DH), lambda i: (i, 0)),
        pl.BlockSpec((BR, DH), lambda i: (i, 0)),
        pl.BlockSpec((BR, DH), lambda i: (i, 0)),
        pl.BlockSpec((BR, 1), lambda i: (i, 0)),
        pl.BlockSpec((BR, D), lambda i: (i, 0)),
        pl.BlockSpec((1, D), lambda i: (0, 0)),
        pl.BlockSpec((1, D), lambda i: (0, 0)),
        pl.BlockSpec((1, D), lambda i: (0, 0)),
    ],
    out_specs=pl.BlockSpec((BR, D), lambda i: (i, 0)),
    out_shape=jax.ShapeDtypeStruct((N, D), jnp.float32),
)


def kernel(x, edge_index, W, b, gamma, beta):
    ei = edge_index.astype(jnp.int32)
    pad = E_PAD - E
    arange_pad = jnp.arange(pad, dtype=jnp.int32)
    pad_src = (arange_pad * 613) % N
    pad_dst = N + arange_pad % (N_PAD - N)
    src_p = jnp.concatenate([ei[0], pad_src]).reshape(NS, NCHUNK, CH)
    dst_p = jnp.concatenate([ei[1], pad_dst]).reshape(NS, NCHUNK, CH)

    deg = _sc_degree(dst_p)
    hp0, hp1, dinv = _mm_call(x, W, deg.reshape(N_PAD, 1)[:N])
    s0, s1 = _sc_segsum(hp0, hp1, src_p, dst_p)
    return _final_call(s0, s1, hp0, hp1, dinv, x, b.reshape(1, D),
                       gamma.reshape(1, D), beta.reshape(1, D))
